# baseline (speedup 1.0000x reference)
import jax
import jax.numpy as jnp
from jax.experimental import pallas as pl
from jax.experimental.pallas import tpu as pltpu

_M_BLK = 512
_N_SUB = 2


def _ffn_kernel(x_ref, w1_ref, w3_ref, w2_ref, o_ref):
    w1 = w1_ref[...]
    w3 = w3_ref[...]
    w2 = w2_ref[...]
    sub = _M_BLK // _N_SUB
    for j in range(_N_SUB):
        rows = pl.ds(j * sub, sub)
        xb = x_ref[rows, :]
        a = jnp.dot(xb, w1, preferred_element_type=jnp.float32)
        b = jnp.dot(xb, w3, preferred_element_type=jnp.float32)
        inter = a * jax.nn.sigmoid(a) * b
        o_ref[rows, :] = jnp.dot(inter, w2, preferred_element_type=jnp.float32)


def kernel(x, gate_W, W1, W3, W2):
    B, S, H = x.shape
    h = x.reshape(-1, H)
    M = h.shape[0]
    w1 = W1[0]
    w3 = W3[0]
    w2 = W2[0]
    F = w1.shape[1]
    out = pl.pallas_call(
        _ffn_kernel,
        grid=(M // _M_BLK,),
        in_specs=[
            pl.BlockSpec((_M_BLK, H), lambda i: (i, 0)),
            pl.BlockSpec((H, F), lambda i: (0, 0)),
            pl.BlockSpec((H, F), lambda i: (0, 0)),
            pl.BlockSpec((F, H), lambda i: (0, 0)),
        ],
        out_specs=pl.BlockSpec((_M_BLK, H), lambda i: (i, 0)),
        out_shape=jax.ShapeDtypeStruct((M, H), jnp.float32),
        compiler_params=pltpu.CompilerParams(
            dimension_semantics=("parallel",),
        ),
    )(h, w1, w3, w2)
    return out.reshape(B, S, H)


# speedup vs baseline: 1.0435x; 1.0435x over previous
import jax
import jax.numpy as jnp
from jax.experimental import pallas as pl

_M_BLK = 256


def _ffn_kernel(x_ref, w1_ref, w3_ref, w2_ref, o_ref):
    xb = x_ref[...].astype(jnp.bfloat16)
    a = jnp.dot(xb, w1_ref[...], preferred_element_type=jnp.float32)
    b = jnp.dot(xb, w3_ref[...], preferred_element_type=jnp.float32)
    inter = (a * jax.nn.sigmoid(a) * b).astype(jnp.bfloat16)
    o_ref[...] = jnp.dot(inter, w2_ref[...], preferred_element_type=jnp.float32)


def kernel(x, gate_W, W1, W3, W2):
    B, S, H = x.shape
    h = x.reshape(-1, H)
    M = h.shape[0]
    w1 = W1[0].astype(jnp.bfloat16)
    w3 = W3[0].astype(jnp.bfloat16)
    w2 = W2[0].astype(jnp.bfloat16)
    F = w1.shape[1]
    out = pl.pallas_call(
        _ffn_kernel,
        grid=(M // _M_BLK,),
        in_specs=[
            pl.BlockSpec((_M_BLK, H), lambda i: (i, 0)),
            pl.BlockSpec((H, F), lambda i: (0, 0)),
            pl.BlockSpec((H, F), lambda i: (0, 0)),
            pl.BlockSpec((F, H), lambda i: (0, 0)),
        ],
        out_specs=pl.BlockSpec((_M_BLK, H), lambda i: (i, 0)),
        out_shape=jax.ShapeDtypeStruct((M, H), jnp.float32),
    )(h, w1, w3, w2)
    return out.reshape(B, S, H)


# speedup vs baseline: 1.0737x; 1.0290x over previous
import jax
import jax.numpy as jnp
from jax.experimental import pallas as pl
from jax.experimental.pallas import tpu as pltpu

_M_BLK = 512


def _ffn_kernel(x_ref, w1_ref, w3_ref, w2_ref, o_ref):
    xb = x_ref[...].astype(jnp.bfloat16)
    a = jnp.dot(xb, w1_ref[...], preferred_element_type=jnp.float32)
    b = jnp.dot(xb, w3_ref[...], preferred_element_type=jnp.float32)
    inter = (a * jax.nn.sigmoid(a) * b).astype(jnp.bfloat16)
    o_ref[...] = jnp.dot(inter, w2_ref[...], preferred_element_type=jnp.float32)


def kernel(x, gate_W, W1, W3, W2):
    B, S, H = x.shape
    h = x.reshape(-1, H)
    M = h.shape[0]
    w1 = W1[0].astype(jnp.bfloat16)
    w3 = W3[0].astype(jnp.bfloat16)
    w2 = W2[0].astype(jnp.bfloat16)
    F = w1.shape[1]
    out = pl.pallas_call(
        _ffn_kernel,
        grid=(M // _M_BLK,),
        in_specs=[
            pl.BlockSpec((_M_BLK, H), lambda i: (i, 0)),
            pl.BlockSpec((H, F), lambda i: (0, 0)),
            pl.BlockSpec((H, F), lambda i: (0, 0)),
            pl.BlockSpec((F, H), lambda i: (0, 0)),
        ],
        out_specs=pl.BlockSpec((_M_BLK, H), lambda i: (i, 0)),
        out_shape=jax.ShapeDtypeStruct((M, H), jnp.float32),
        compiler_params=pltpu.CompilerParams(
            vmem_limit_bytes=128 * 1024 * 1024,
        ),
    )(h, w1, w3, w2)
    return out.reshape(B, S, H)
